# Initial kernel scaffold; baseline (speedup 1.0000x reference)
import jax
import jax.numpy as jnp
from jax.experimental import pallas as pl


def kernel(x, edge_index, edge_weight, batch, W1, b1, W2, b2, W3, b3, ln_g, ln_b, fcW, fcb, w_mu, w_rho, b_mu, b_rho):
    raise NotImplementedError("write your pallas kernel here")


# speedup vs baseline: 13.4666x; 13.4666x over previous
import functools

import jax
import jax.numpy as jnp
from jax import lax
from jax.experimental import pallas as pl
from jax.experimental.pallas import tpu as pltpu
from jax.experimental.pallas import tpu_sc as plsc

N = 10000
E = 320000
NG = 64

NUM_CORES = 2
NUM_SUBCORES = 16
NUM_WORKERS = NUM_CORES * NUM_SUBCORES
EPT = E // NUM_WORKERS
ROWS_PER_TILE = N // NUM_SUBCORES


def _sc_mesh():
    return plsc.VectorSubcoreMesh(core_axis_name="c", subcore_axis_name="s")


DEG_CH = 2000
DEG_NCH = EPT // DEG_CH


@functools.partial(
    pl.kernel,
    mesh=_sc_mesh(),
    out_type=jax.ShapeDtypeStruct((2 * N,), jnp.float32),
    scratch_types=[
        pltpu.VMEM((DEG_CH,), jnp.int32),
        pltpu.VMEM((DEG_CH,), jnp.float32),
        pltpu.VMEM_SHARED((N,), jnp.float32),
    ],
)
def _deg_kernel(c_hbm, w_hbm, out_hbm, c_v, w_v, acc):
    cid = lax.axis_index("c")
    sid = lax.axis_index("s")

    zero = jnp.zeros((16,), jnp.float32)

    def zf(i, _):
        w_v[pl.ds(i * 16, 16)] = zero
        return 0

    lax.fori_loop(0, DEG_CH // 16, zf, 0)

    @pl.when(sid < 10)
    def _():
        pltpu.sync_copy(w_v.at[pl.ds(0, 1000)], acc.at[pl.ds(sid * 1000, 1000)])

    plsc.subcore_barrier()

    base0 = cid * (E // 2) + sid * EPT
    for i in range(DEG_NCH):
        base = base0 + i * DEG_CH
        pltpu.sync_copy(c_hbm.at[pl.ds(base, DEG_CH)], c_v)
        pltpu.sync_copy(w_hbm.at[pl.ds(base, DEG_CH)], w_v)
        pltpu.sync_copy(w_v, acc.at[c_v], add=True)

    plsc.subcore_barrier()

    @pl.when(sid < 10)
    def _():
        pltpu.sync_copy(acc.at[pl.ds(sid * 1000, 1000)], w_v.at[pl.ds(0, 1000)])
        pltpu.sync_copy(w_v.at[pl.ds(0, 1000)],
                        out_hbm.at[pl.ds(cid * N + sid * 1000, 1000)])


def _make_edge_pass(H, CH):
    nch = EPT // CH
    assert nch * CH == EPT and CH % 16 == 0

    params = (None if H == 128 else
              pltpu.CompilerParams(use_tc_tiling_on_sc=False))

    @functools.partial(
        pl.kernel,
        mesh=_sc_mesh(),
        compiler_params=params,
        out_type=jax.ShapeDtypeStruct((2 * N, H), jnp.float32),
        scratch_types=[
            pltpu.VMEM((CH,), jnp.int32),
            pltpu.VMEM((CH,), jnp.int32),
            pltpu.VMEM((CH,), jnp.float32),
            pltpu.VMEM((CH, H), jnp.float32),
            pltpu.VMEM_SHARED((N, H), jnp.float32),
            pltpu.SemaphoreType.DMA,
        ],
    )
    def edge_pass(y_hbm, r_hbm, c_hbm, w_hbm, out_hbm, r_v, c_v, w_v, rows_v,
                  acc, sem):
        cid = lax.axis_index("c")
        sid = lax.axis_index("s")

        zero = jnp.zeros((16,), jnp.float32)

        def zf(j, _):
            for k in range(H // 16):
                rows_v[j, pl.ds(k * 16, 16)] = zero
            return 0

        zrows = 200 if CH >= 200 else 40
        nzc = 1000 // zrows
        lax.fori_loop(0, zrows, zf, 0)

        @pl.when(sid < 10)
        def _():
            for p in range(nzc):
                pltpu.sync_copy(
                    rows_v.at[pl.ds(0, zrows)],
                    acc.at[pl.ds(sid * 1000 + p * zrows, zrows)])

        plsc.subcore_barrier()

        base0 = cid * (E // 2) + sid * EPT

        def chunk_body(i, _):
            base = pl.multiple_of(base0 + i * CH, 8)
            pltpu.sync_copy(r_hbm.at[pl.ds(base, CH)], r_v)
            pltpu.sync_copy(c_hbm.at[pl.ds(base, CH)], c_v)
            pltpu.sync_copy(w_hbm.at[pl.ds(base, CH)], w_v)
            pltpu.async_copy(y_hbm.at[r_v], rows_v, sem).wait()

            def sbody(g, _):
                j0 = g * 16
                wv = w_v[pl.ds(j0, 16)]
                for l in range(16):
                    wl = wv[l]
                    for k in range(H // 16):
                        sl = pl.ds(k * 16, 16)
                        rows_v[j0 + l, sl] = rows_v[j0 + l, sl] * wl
                return 0

            lax.fori_loop(0, CH // 16, sbody, 0)
            pltpu.sync_copy(rows_v, acc.at[c_v], add=True)
            return 0

        lax.fori_loop(0, nch, chunk_body, 0)

        plsc.subcore_barrier()

        @pl.when(sid < 10)
        def _():
            for p in range(nzc):
                rbase = sid * 1000 + p * zrows
                pltpu.sync_copy(acc.at[pl.ds(rbase, zrows)],
                                rows_v.at[pl.ds(0, zrows)])
                pltpu.sync_copy(rows_v.at[pl.ds(0, zrows)],
                                out_hbm.at[pl.ds(cid * N + rbase, zrows)])

    return edge_pass


_edge_pass_128 = _make_edge_pass(128, 80)
_edge_pass_64 = _make_edge_pass(64, 400)
_edge_pass_32 = _make_edge_pass(32, 2000)


def _tc_first(deg_ref, x_ref, w1_ref, y_ref, dinv_ref):
    deg = deg_ref[0] + deg_ref[1] + 1.0
    dinv = lax.rsqrt(deg)
    dinv_ref[...] = dinv
    xw = jnp.dot(x_ref[...], w1_ref[...], preferred_element_type=jnp.float32, precision=lax.Precision.HIGHEST)
    y_ref[...] = xw * dinv


def _tc_mid(acc_ref, y_ref, dinv_ref, b_ref, w_ref, yout_ref):
    dinv = dinv_ref[...]
    h = dinv * (acc_ref[0] + acc_ref[1] + y_ref[...]) + b_ref[...]
    h = jnp.where(h > 0, h, jnp.exp(h) - 1.0)
    xw = jnp.dot(h, w_ref[...], preferred_element_type=jnp.float32, precision=lax.Precision.HIGHEST)
    yout_ref[...] = xw * dinv


def _tc_head(acc_ref, y_ref, dinv_ref, b_ref, batch_ref, lng_ref, lnb_ref,
             fcw_ref, fcb_ref, wmu_ref, wrho_ref, bmu_ref, brho_ref,
             epsw_ref, epsb_ref, out_ref):
    h = dinv_ref[...] * (acc_ref[0] + acc_ref[1] + y_ref[...]) + b_ref[...]
    h = jnp.where(h > 0, h, jnp.exp(h) - 1.0)
    gids = lax.broadcasted_iota(jnp.int32, (NG, N), 0)
    m = (gids == batch_ref[...]).astype(jnp.float32)
    sums = jnp.dot(m, h, preferred_element_type=jnp.float32, precision=lax.Precision.HIGHEST)
    cnt = jnp.sum(m, axis=1, keepdims=True)
    pooled = sums / jnp.maximum(cnt, 1.0)
    mu = jnp.mean(pooled, axis=1, keepdims=True)
    var = jnp.mean((pooled - mu) ** 2, axis=1, keepdims=True)
    ln = (pooled - mu) / jnp.sqrt(var + 1e-5) * lng_ref[...] + lnb_ref[...]
    h2 = jnp.dot(ln, fcw_ref[...], preferred_element_type=jnp.float32, precision=lax.Precision.HIGHEST)
    h2 = h2 + fcb_ref[...]
    h2 = jnp.where(h2 > 0, h2, jnp.exp(h2) - 1.0)
    wgt = wmu_ref[...] + jnp.log1p(jnp.exp(wrho_ref[...])) * epsw_ref[...]
    bia = bmu_ref[...] + jnp.log1p(jnp.exp(brho_ref[...])) * epsb_ref[...]
    out_ref[...] = jnp.sum(h2 * wgt, axis=1, keepdims=True) + bia


def _call_tc(body, out_shapes, *args):
    return pl.pallas_call(
        body,
        out_shape=out_shapes,
    )(*args)


def kernel(x, edge_index, edge_weight, batch, W1, b1, W2, b2, W3, b3,
           ln_g, ln_b, fcW, fcb, w_mu, w_rho, b_mu, b_rho):
    row = edge_index[0]
    col = edge_index[1]

    deg2 = _deg_kernel(col, edge_weight)
    degp = deg2.reshape(2, N, 1)

    y1, dinv = _call_tc(
        _tc_first,
        (jax.ShapeDtypeStruct((N, 128), jnp.float32),
         jax.ShapeDtypeStruct((N, 1), jnp.float32)),
        degp, x, W1)

    acc1 = _edge_pass_128(y1, row, col, edge_weight).reshape(2, N, 128)
    y2 = _call_tc(
        _tc_mid, jax.ShapeDtypeStruct((N, 64), jnp.float32),
        acc1, y1, dinv, b1.reshape(1, 128), W2)

    acc2 = _edge_pass_64(y2, row, col, edge_weight).reshape(2, N, 64)
    y3 = _call_tc(
        _tc_mid, jax.ShapeDtypeStruct((N, 32), jnp.float32),
        acc2, y2, dinv, b2.reshape(1, 64), W3)

    acc3 = _edge_pass_32(y3, row, col, edge_weight).reshape(2, N, 32)

    kk = jax.random.key(42)
    eps_w = jax.random.normal(jax.random.fold_in(kk, 1), (1, 8), jnp.float32)
    eps_b = jax.random.normal(jax.random.fold_in(kk, 2), (1,), jnp.float32)

    out = _call_tc(
        _tc_head, jax.ShapeDtypeStruct((NG, 1), jnp.float32),
        acc3, y3, dinv, b3.reshape(1, 32), batch.reshape(1, N),
        ln_g.reshape(1, 32), ln_b.reshape(1, 32), fcW, fcb.reshape(1, 8),
        w_mu, w_rho, b_mu.reshape(1, 1), b_rho.reshape(1, 1),
        eps_w, eps_b.reshape(1, 1))
    return out
